# baseline (speedup 1.0000x reference)
import functools

import jax
import jax.numpy as jnp
from jax import lax
from jax.experimental import pallas as pl
from jax.experimental.pallas import tpu as pltpu
from jax.experimental.pallas import tpu_sc as plsc

_info = plsc.get_sparse_core_info()
_NC, _NS, _L = _info.num_cores, _info.num_subcores, _info.num_lanes
_NW = _NC * _NS

_CHUNK = 64


def _build(b, s, d):
    assert s == _NW * _CHUNK
    mesh = plsc.VectorSubcoreMesh(core_axis_name="c", subcore_axis_name="s")

    @functools.partial(
        pl.kernel,
        mesh=mesh,
        out_type=jax.ShapeDtypeStruct((b, s, d), jnp.float32),
        scratch_types=[
            pltpu.VMEM((b, _CHUNK), jnp.int32),
            pltpu.VMEM((b, _CHUNK, d), jnp.float32),
            pltpu.VMEM_SHARED((_NS, _CHUNK, d), jnp.float32),
            pltpu.SemaphoreType.DMA,
            pltpu.SemaphoreType.DMA,
            pltpu.SemaphoreType.DMA,
        ],
    )
    def k(x_hbm, table_hbm, pos_hbm, out_hbm, idx_v, rows_v, pos_sh,
          sem_p, sem_g, sem_w):
        sid = lax.axis_index("s")
        wid = sid * _NC + lax.axis_index("c")
        s0 = wid * _CHUNK
        i_cps = [
            pltpu.async_copy(
                x_hbm.at[j, pl.ds(s0, _CHUNK)], idx_v.at[j], sem_p
            )
            for j in range(b)
        ]
        pos_cp = pltpu.async_copy(pos_hbm.at[0, pl.ds(s0, _CHUNK)],
                                  pos_sh.at[sid], sem_p)
        with jax.named_scope("wait_pos"):
            pos_cp.wait()
        r_cps = [
            pltpu.async_copy(pos_sh.at[sid], rows_v.at[j], sem_p)
            for j in range(b)
        ]
        with jax.named_scope("wait_idx"):
            for cp in i_cps:
                cp.wait()
        g_cps = []
        for j in range(b):
            with jax.named_scope(f"wait_rep{j}"):
                r_cps[j].wait()
            g_cps.append(
                pltpu.async_copy(
                    table_hbm.at[idx_v.at[j]], rows_v.at[j], sem_g, add=True
                )
            )
        w_cps = []
        for j in range(b):
            with jax.named_scope(f"wait_g{j}"):
                g_cps[j].wait()
            w_cps.append(
                pltpu.async_copy(
                    rows_v.at[j], out_hbm.at[j, pl.ds(s0, _CHUNK)], sem_w
                )
            )
        with jax.named_scope("wait_wb"):
            for cp in w_cps:
                cp.wait()

    return k


def kernel(X, token_table, pos_embed):
    b, s = X.shape
    vocab, d = token_table.shape
    return _build(b, s, d)(X.astype(jnp.int32), token_table, pos_embed)
